# baseline (speedup 1.0000x reference)
import functools

import jax
import jax.numpy as jnp
import numpy as np
from jax.experimental import pallas as pl
from jax.experimental.pallas import tpu as pltpu
from jax.experimental.pallas import tpu_sc as plsc

_VEL_W, _CON_W, _BND_W, _OBS_W, _DIV_W = 1.0, 0.1, 0.5, 1.0, 0.1
_B, _N, _M = 4, 2048, 16
_K = 5
_R = 512
_NB = _N // _R
_S = 100
_OBW = 128
_NW = 32
_LANES = 16
_GPAD = _NW * _LANES


def _div_indices():
    rng = np.random.default_rng(0)
    idx = np.stack([rng.permutation(_N)[:4] for _ in range(_S)])
    gidx = np.zeros((3, _GPAD), np.int32)
    for g in range(_B * _S):
        b, s = divmod(g, _S)
        for j in range(3):
            gidx[j, g] = b * _N + idx[s, j]
    return gidx


_DIV_IDX = _div_indices()

_sc_mesh = plsc.VectorSubcoreMesh(core_axis_name="c", subcore_axis_name="s")


@functools.partial(
    pl.kernel,
    mesh=_sc_mesh,
    out_type=jax.ShapeDtypeStruct((_NW, _LANES), jnp.float32),
    scratch_types=[
        pltpu.VMEM((_LANES,), jnp.int32),
        pltpu.VMEM((_LANES,), jnp.int32),
        pltpu.VMEM((_LANES,), jnp.int32),
        pltpu.VMEM((_LANES,), jnp.float32),
        pltpu.VMEM((_LANES,), jnp.float32),
        pltpu.VMEM((_LANES,), jnp.float32),
        pltpu.VMEM((_LANES,), jnp.float32),
        pltpu.VMEM((_LANES,), jnp.float32),
        pltpu.VMEM((_LANES,), jnp.float32),
        pltpu.VMEM((_LANES,), jnp.float32),
        pltpu.VMEM((_LANES,), jnp.float32),
        pltpu.VMEM((_LANES,), jnp.float32),
        pltpu.SemaphoreType.DMA,
    ],
)
def _sc_divergence(px_hbm, py_hbm, vx_hbm, vy_hbm, idx0_hbm, idx1_hbm,
                   idx2_hbm, out_hbm, i0_v, i1_v, i2_v, p0x_v, p0y_v,
                   v0x_v, v0y_v, p1x_v, v1x_v, p2y_v, v2y_v, res_v, sem):
    wid = jax.lax.axis_index("s") * 2 + jax.lax.axis_index("c")
    base = wid * _LANES
    pltpu.sync_copy(idx0_hbm.at[pl.ds(base, _LANES)], i0_v)
    pltpu.sync_copy(idx1_hbm.at[pl.ds(base, _LANES)], i1_v)
    pltpu.sync_copy(idx2_hbm.at[pl.ds(base, _LANES)], i2_v)
    pltpu.async_copy(px_hbm.at[i0_v], p0x_v, sem).wait()
    pltpu.async_copy(py_hbm.at[i0_v], p0y_v, sem).wait()
    pltpu.async_copy(vx_hbm.at[i0_v], v0x_v, sem).wait()
    pltpu.async_copy(vy_hbm.at[i0_v], v0y_v, sem).wait()
    pltpu.async_copy(px_hbm.at[i1_v], p1x_v, sem).wait()
    pltpu.async_copy(vx_hbm.at[i1_v], v1x_v, sem).wait()
    pltpu.async_copy(py_hbm.at[i2_v], p2y_v, sem).wait()
    pltpu.async_copy(vy_hbm.at[i2_v], v2y_v, sem).wait()
    dxs = p1x_v[...] - p0x_v[...]
    dys = p2y_v[...] - p0y_v[...]
    dvx = v1x_v[...] - v0x_v[...]
    dvy = v2y_v[...] - v0y_v[...]
    div = dvx / (dxs + 1e-6) + dvy / (dys + 1e-6)
    res_v[...] = div * div
    pltpu.sync_copy(res_v, out_hbm.at[wid])


def _loss_body(rows_ref, cols_ref, obs_ref, out_ref):
    i = pl.program_id(1)
    rows = rows_ref[0]
    px_i = rows[:, 0:1]
    py_i = rows[:, 1:2]
    vx_i = rows[:, 2:3]
    vy_i = rows[:, 3:4]
    tx_i = rows[:, 4:5]
    ty_i = rows[:, 5:6]
    msk = rows[:, 6:7]

    cols = cols_ref[0]
    px_j = cols[0:1, :]
    py_j = cols[1:2, :]
    vx_j = cols[2:3, :]
    vy_j = cols[3:4, :]

    dx = px_i - px_j
    dy = py_i - py_j
    d = jnp.sqrt(dx * dx + dy * dy + 1e-12)
    wx = vx_i - vx_j
    wy = vy_i - vy_j
    vsq = wx * wx + wy * wy

    col_ids = jax.lax.broadcasted_iota(jnp.int32, (1, _N), 1)
    row_ids = i * _R + jax.lax.broadcasted_iota(jnp.int32, (_R, 1), 0)
    big = jnp.float32(1e6)
    dns = jnp.where(col_ids == row_ids, big, d)

    acc = jnp.zeros((_R, 1), jnp.float32)
    for _ in range(_K):
        dmin = jnp.min(dns, axis=1, keepdims=True)
        eq = dns == dmin
        jmin = jnp.min(jnp.where(eq, col_ids, jnp.int32(_N)),
                       axis=1, keepdims=True)
        sel = col_ids == jmin
        vsel = jnp.sum(jnp.where(sel, vsq, 0.0), axis=1, keepdims=True)
        vd = jnp.sqrt(vsel + 1e-12)
        acc = acc + vd * (1.0 / (dmin + 1e-6))
        dns = jnp.where(sel, big, dns)
    con_part = jnp.sum(acc)

    vl_part = jnp.sum((vx_i - tx_i) ** 2 + (vy_i - ty_i) ** 2)

    a0, a1, a2, a3 = px_i, 1.0 - px_i, py_i, 1.0 - py_i
    is0 = (a0 <= a1) & (a0 <= a2) & (a0 <= a3)
    is1 = (~is0) & (a1 <= a2) & (a1 <= a3)
    is2 = (~is0) & (~is1) & (a2 <= a3)
    is3 = (~is0) & (~is1) & (~is2)
    nx = jnp.where(is0, -1.0, jnp.where(is1, 1.0, 0.0))
    ny = jnp.where(is2, -1.0, jnp.where(is3, 1.0, 0.0))
    nc = vx_i * nx + vy_i * ny
    bl_num = jnp.sum(nc * nc * msk)
    bl_cnt = jnp.sum(msk)

    cx = obs_ref[0, 0:1, :]
    cy = obs_ref[0, 1:2, :]
    rr = obs_ref[0, 2:3, :]
    dxo = px_i - cx
    dyo = py_i - cy
    disto = jnp.sqrt(dxo * dxo + dyo * dyo + 1e-12)
    near = (disto < rr * 2.0).astype(jnp.float32)
    wexp = jnp.exp(-(disto - rr) / (rr * 0.5))
    proj = (vx_i * dxo + vy_i * dyo) / (disto + 1e-6)
    pen = wexp * jnp.maximum(-proj, 0.0) ** 2
    pns = jnp.sum(pen * near, axis=0, keepdims=True)
    ncnt = jnp.sum(near, axis=0, keepdims=True)

    def bc(s):
        return jnp.broadcast_to(jnp.reshape(s, (1, 1)), (1, 128))

    tile = jnp.concatenate(
        [bc(con_part), bc(vl_part), bc(bl_num), bc(bl_cnt),
         pns, ncnt, jnp.zeros((2, 128), jnp.float32)], axis=0)
    out_ref[0, 0] = tile


@jax.jit
def kernel(predicted_velocities, target_velocities, positions, obstacles,
           boundary_mask):
    px = positions[..., 0].reshape(-1)
    py = positions[..., 1].reshape(-1)
    vx = predicted_velocities[..., 0].reshape(-1)
    vy = predicted_velocities[..., 1].reshape(-1)
    idx0 = jnp.asarray(_DIV_IDX[0])
    idx1 = jnp.asarray(_DIV_IDX[1])
    idx2 = jnp.asarray(_DIV_IDX[2])
    div_sq = _sc_divergence(px, py, vx, vy, idx0, idx1, idx2)
    dl = div_sq.reshape(-1)[: _B * _S].sum() / (_B * _S)

    mask_f = boundary_mask.astype(jnp.float32)[..., None]
    zeros_rows = jnp.zeros((_B, _N, 1), jnp.float32)
    rows = jnp.concatenate(
        [positions, predicted_velocities, target_velocities, mask_f,
         zeros_rows], axis=-1)
    cols = jnp.concatenate(
        [jnp.transpose(positions, (0, 2, 1)),
         jnp.transpose(predicted_velocities, (0, 2, 1)),
         jnp.zeros((_B, 4, _N), jnp.float32)], axis=1)
    obs_p = jnp.zeros((_B, 8, _OBW), jnp.float32)
    obs_p = obs_p.at[:, 0:3, 0:_M].set(jnp.transpose(obstacles, (0, 2, 1)))

    parts = pl.pallas_call(
        _loss_body,
        grid=(_B, _NB),
        in_specs=[
            pl.BlockSpec((1, _R, 8), lambda b, i: (b, i, 0)),
            pl.BlockSpec((1, 8, _N), lambda b, i: (b, 0, 0)),
            pl.BlockSpec((1, 8, _OBW), lambda b, i: (b, 0, 0)),
        ],
        out_specs=pl.BlockSpec((1, 1, 8, 128), lambda b, i: (b, i, 0, 0)),
        out_shape=jax.ShapeDtypeStruct((_B, _NB, 8, 128), jnp.float32),
    )(rows, cols, obs_p)

    cl = parts[:, :, 0, 0].sum() / (_B * _N * _K)
    vl = parts[:, :, 1, 0].sum() / (_B * _N * 2)
    bln = parts[:, :, 2, 0].sum()
    blc = parts[:, :, 3, 0].sum()
    bl = jnp.where(blc > 0, bln / jnp.maximum(blc, 1.0), 0.0)
    pns = parts[:, :, 4, 0:_M].sum(axis=1)
    cnt = parts[:, :, 5, 0:_M].sum(axis=1)
    rr = obstacles[:, :, 2]
    penalty = jnp.where(cnt > 0, pns / jnp.maximum(cnt, 1.0), 0.0)
    penalty = penalty * (rr > 0).astype(jnp.float32)
    ol = penalty.sum() / (_B * _M)

    return (_VEL_W * vl + _CON_W * cl + _BND_W * bl + _OBS_W * ol
            + _DIV_W * dl)


# speedup vs baseline: 1.2107x; 1.2107x over previous
import functools

import jax
import jax.numpy as jnp
import numpy as np
from jax.experimental import pallas as pl
from jax.experimental.pallas import tpu as pltpu
from jax.experimental.pallas import tpu_sc as plsc

_VEL_W, _CON_W, _BND_W, _OBS_W, _DIV_W = 1.0, 0.1, 0.5, 1.0, 0.1
_B, _N, _M = 4, 2048, 16
_K = 5
_R = 512
_NB = _N // _R
_S = 100
_OBW = 128
_NW = 32
_LANES = 16
_GPAD = _NW * _LANES


def _div_indices():
    rng = np.random.default_rng(0)
    idx = np.stack([rng.permutation(_N)[:4] for _ in range(_S)])
    gidx = np.zeros((3, _GPAD), np.int32)
    for g in range(_B * _S):
        b, s = divmod(g, _S)
        for j in range(3):
            gidx[j, g] = b * _N + idx[s, j]
    return gidx


_DIV_IDX = _div_indices()

_sc_mesh = plsc.VectorSubcoreMesh(core_axis_name="c", subcore_axis_name="s")


@functools.partial(
    pl.kernel,
    mesh=_sc_mesh,
    out_type=jax.ShapeDtypeStruct((_NW, _LANES), jnp.float32),
    scratch_types=[
        pltpu.VMEM((_LANES,), jnp.int32),
        pltpu.VMEM((_LANES,), jnp.int32),
        pltpu.VMEM((_LANES,), jnp.int32),
        pltpu.VMEM((_LANES,), jnp.float32),
        pltpu.VMEM((_LANES,), jnp.float32),
        pltpu.VMEM((_LANES,), jnp.float32),
        pltpu.VMEM((_LANES,), jnp.float32),
        pltpu.VMEM((_LANES,), jnp.float32),
        pltpu.VMEM((_LANES,), jnp.float32),
        pltpu.VMEM((_LANES,), jnp.float32),
        pltpu.VMEM((_LANES,), jnp.float32),
        pltpu.VMEM((_LANES,), jnp.float32),
        pltpu.SemaphoreType.DMA,
    ],
)
def _sc_divergence(px_hbm, py_hbm, vx_hbm, vy_hbm, idx0_hbm, idx1_hbm,
                   idx2_hbm, out_hbm, i0_v, i1_v, i2_v, p0x_v, p0y_v,
                   v0x_v, v0y_v, p1x_v, v1x_v, p2y_v, v2y_v, res_v, sem):
    wid = jax.lax.axis_index("s") * 2 + jax.lax.axis_index("c")
    base = wid * _LANES
    pltpu.sync_copy(idx0_hbm.at[pl.ds(base, _LANES)], i0_v)
    pltpu.sync_copy(idx1_hbm.at[pl.ds(base, _LANES)], i1_v)
    pltpu.sync_copy(idx2_hbm.at[pl.ds(base, _LANES)], i2_v)
    copies = [
        pltpu.async_copy(px_hbm.at[i0_v], p0x_v, sem),
        pltpu.async_copy(py_hbm.at[i0_v], p0y_v, sem),
        pltpu.async_copy(vx_hbm.at[i0_v], v0x_v, sem),
        pltpu.async_copy(vy_hbm.at[i0_v], v0y_v, sem),
        pltpu.async_copy(px_hbm.at[i1_v], p1x_v, sem),
        pltpu.async_copy(vx_hbm.at[i1_v], v1x_v, sem),
        pltpu.async_copy(py_hbm.at[i2_v], p2y_v, sem),
        pltpu.async_copy(vy_hbm.at[i2_v], v2y_v, sem),
    ]
    for c in copies:
        c.wait()
    dxs = p1x_v[...] - p0x_v[...]
    dys = p2y_v[...] - p0y_v[...]
    dvx = v1x_v[...] - v0x_v[...]
    dvy = v2y_v[...] - v0y_v[...]
    div = dvx / (dxs + 1e-6) + dvy / (dys + 1e-6)
    res_v[...] = div * div
    pltpu.sync_copy(res_v, out_hbm.at[wid])


def _loss_body(rows_ref, cols_ref, obs_ref, out_ref):
    i = pl.program_id(1)
    rows = rows_ref[0]
    px_i = rows[:, 0:1]
    py_i = rows[:, 1:2]
    vx_i = rows[:, 2:3]
    vy_i = rows[:, 3:4]
    tx_i = rows[:, 4:5]
    ty_i = rows[:, 5:6]
    msk = rows[:, 6:7]

    cols = cols_ref[0]
    px_j = cols[0:1, :]
    py_j = cols[1:2, :]
    vx_j = cols[2:3, :]
    vy_j = cols[3:4, :]

    dx = px_i - px_j
    dy = py_i - py_j
    d = jnp.sqrt(dx * dx + dy * dy + 1e-12)
    wx = vx_i - vx_j
    wy = vy_i - vy_j
    vsq = wx * wx + wy * wy

    col_ids = jax.lax.broadcasted_iota(jnp.int32, (1, _N), 1)
    row_ids = i * _R + jax.lax.broadcasted_iota(jnp.int32, (_R, 1), 0)
    idmask = jnp.int32(-2048)
    kbits = (jax.lax.bitcast_convert_type(d, jnp.int32) & idmask) | col_ids
    bigbits = jnp.int32(0x7F000000)
    kbits = jnp.where(col_ids == row_ids, bigbits, kbits)
    keys = jax.lax.bitcast_convert_type(kbits, jnp.float32)
    bigkey = jax.lax.bitcast_convert_type(bigbits, jnp.float32)

    acc = jnp.zeros((_R, 1), jnp.float32)
    for _ in range(_K):
        kmin = jnp.min(keys, axis=1, keepdims=True)
        sel = keys == kmin
        vsel = jnp.sum(jnp.where(sel, vsq, 0.0), axis=1, keepdims=True)
        vd = jnp.sqrt(vsel + 1e-12)
        dmin = jax.lax.bitcast_convert_type(
            jax.lax.bitcast_convert_type(kmin, jnp.int32) & idmask,
            jnp.float32)
        acc = acc + vd * (1.0 / (dmin + 1e-6))
        keys = jnp.where(sel, bigkey, keys)
    con_part = jnp.sum(acc)

    vl_part = jnp.sum((vx_i - tx_i) ** 2 + (vy_i - ty_i) ** 2)

    a0, a1, a2, a3 = px_i, 1.0 - px_i, py_i, 1.0 - py_i
    is0 = (a0 <= a1) & (a0 <= a2) & (a0 <= a3)
    is1 = (~is0) & (a1 <= a2) & (a1 <= a3)
    is2 = (~is0) & (~is1) & (a2 <= a3)
    is3 = (~is0) & (~is1) & (~is2)
    nx = jnp.where(is0, -1.0, jnp.where(is1, 1.0, 0.0))
    ny = jnp.where(is2, -1.0, jnp.where(is3, 1.0, 0.0))
    nc = vx_i * nx + vy_i * ny
    bl_num = jnp.sum(nc * nc * msk)
    bl_cnt = jnp.sum(msk)

    cx = obs_ref[0, 0:1, :]
    cy = obs_ref[0, 1:2, :]
    rr = obs_ref[0, 2:3, :]
    dxo = px_i - cx
    dyo = py_i - cy
    disto = jnp.sqrt(dxo * dxo + dyo * dyo + 1e-12)
    near = (disto < rr * 2.0).astype(jnp.float32)
    wexp = jnp.exp(-(disto - rr) / (rr * 0.5))
    proj = (vx_i * dxo + vy_i * dyo) / (disto + 1e-6)
    pen = wexp * jnp.maximum(-proj, 0.0) ** 2
    pns = jnp.sum(pen * near, axis=0, keepdims=True)
    ncnt = jnp.sum(near, axis=0, keepdims=True)

    def bc(s):
        return jnp.broadcast_to(jnp.reshape(s, (1, 1)), (1, 128))

    tile = jnp.concatenate(
        [bc(con_part), bc(vl_part), bc(bl_num), bc(bl_cnt),
         pns, ncnt, jnp.zeros((2, 128), jnp.float32)], axis=0)
    out_ref[0, 0] = tile


@jax.jit
def kernel(predicted_velocities, target_velocities, positions, obstacles,
           boundary_mask):
    px = positions[..., 0].reshape(-1)
    py = positions[..., 1].reshape(-1)
    vx = predicted_velocities[..., 0].reshape(-1)
    vy = predicted_velocities[..., 1].reshape(-1)
    idx0 = jnp.asarray(_DIV_IDX[0])
    idx1 = jnp.asarray(_DIV_IDX[1])
    idx2 = jnp.asarray(_DIV_IDX[2])
    div_sq = _sc_divergence(px, py, vx, vy, idx0, idx1, idx2)
    dl = div_sq.reshape(-1)[: _B * _S].sum() / (_B * _S)

    mask_f = boundary_mask.astype(jnp.float32)[..., None]
    zeros_rows = jnp.zeros((_B, _N, 1), jnp.float32)
    rows = jnp.concatenate(
        [positions, predicted_velocities, target_velocities, mask_f,
         zeros_rows], axis=-1)
    cols = jnp.concatenate(
        [jnp.transpose(positions, (0, 2, 1)),
         jnp.transpose(predicted_velocities, (0, 2, 1)),
         jnp.zeros((_B, 4, _N), jnp.float32)], axis=1)
    obs_p = jnp.zeros((_B, 8, _OBW), jnp.float32)
    obs_p = obs_p.at[:, 0:3, 0:_M].set(jnp.transpose(obstacles, (0, 2, 1)))

    parts = pl.pallas_call(
        _loss_body,
        grid=(_B, _NB),
        in_specs=[
            pl.BlockSpec((1, _R, 8), lambda b, i: (b, i, 0)),
            pl.BlockSpec((1, 8, _N), lambda b, i: (b, 0, 0)),
            pl.BlockSpec((1, 8, _OBW), lambda b, i: (b, 0, 0)),
        ],
        out_specs=pl.BlockSpec((1, 1, 8, 128), lambda b, i: (b, i, 0, 0)),
        out_shape=jax.ShapeDtypeStruct((_B, _NB, 8, 128), jnp.float32),
    )(rows, cols, obs_p)

    cl = parts[:, :, 0, 0].sum() / (_B * _N * _K)
    vl = parts[:, :, 1, 0].sum() / (_B * _N * 2)
    bln = parts[:, :, 2, 0].sum()
    blc = parts[:, :, 3, 0].sum()
    bl = jnp.where(blc > 0, bln / jnp.maximum(blc, 1.0), 0.0)
    pns = parts[:, :, 4, 0:_M].sum(axis=1)
    cnt = parts[:, :, 5, 0:_M].sum(axis=1)
    rr = obstacles[:, :, 2]
    penalty = jnp.where(cnt > 0, pns / jnp.maximum(cnt, 1.0), 0.0)
    penalty = penalty * (rr > 0).astype(jnp.float32)
    ol = penalty.sum() / (_B * _M)

    return (_VEL_W * vl + _CON_W * cl + _BND_W * bl + _OBS_W * ol
            + _DIV_W * dl)


# speedup vs baseline: 1.2841x; 1.0606x over previous
import functools

import jax
import jax.numpy as jnp
import numpy as np
from jax.experimental import pallas as pl
from jax.experimental.pallas import tpu as pltpu
from jax.experimental.pallas import tpu_sc as plsc

_VEL_W, _CON_W, _BND_W, _OBS_W, _DIV_W = 1.0, 0.1, 0.5, 1.0, 0.1
_B, _N, _M = 4, 2048, 16
_K = 5
_R = 512
_NB = _N // _R
_S = 100
_OBW = 128
_NW = 32
_LANES = 16
_GPAD = _NW * _LANES


def _div_indices():
    rng = np.random.default_rng(0)
    idx = np.stack([rng.permutation(_N)[:4] for _ in range(_S)])
    gidx = np.zeros((3, _GPAD), np.int32)
    for g in range(_B * _S):
        b, s = divmod(g, _S)
        for j in range(3):
            gidx[j, g] = b * _N + idx[s, j]
    return gidx


_DIV_IDX = _div_indices()

_sc_mesh = plsc.VectorSubcoreMesh(core_axis_name="c", subcore_axis_name="s")


@functools.partial(
    pl.kernel,
    mesh=_sc_mesh,
    out_type=jax.ShapeDtypeStruct((_NW, _LANES), jnp.float32),
    scratch_types=[
        pltpu.VMEM((_LANES,), jnp.int32),
        pltpu.VMEM((_LANES,), jnp.int32),
        pltpu.VMEM((_LANES,), jnp.int32),
        pltpu.VMEM((_LANES,), jnp.float32),
        pltpu.VMEM((_LANES,), jnp.float32),
        pltpu.VMEM((_LANES,), jnp.float32),
        pltpu.VMEM((_LANES,), jnp.float32),
        pltpu.VMEM((_LANES,), jnp.float32),
        pltpu.VMEM((_LANES,), jnp.float32),
        pltpu.VMEM((_LANES,), jnp.float32),
        pltpu.VMEM((_LANES,), jnp.float32),
        pltpu.VMEM((_LANES,), jnp.float32),
        pltpu.SemaphoreType.DMA,
    ],
)
def _sc_divergence(px_hbm, py_hbm, vx_hbm, vy_hbm, idx0_hbm, idx1_hbm,
                   idx2_hbm, out_hbm, i0_v, i1_v, i2_v, p0x_v, p0y_v,
                   v0x_v, v0y_v, p1x_v, v1x_v, p2y_v, v2y_v, res_v, sem):
    wid = jax.lax.axis_index("s") * 2 + jax.lax.axis_index("c")
    base = wid * _LANES
    pltpu.sync_copy(idx0_hbm.at[pl.ds(base, _LANES)], i0_v)
    pltpu.sync_copy(idx1_hbm.at[pl.ds(base, _LANES)], i1_v)
    pltpu.sync_copy(idx2_hbm.at[pl.ds(base, _LANES)], i2_v)
    copies = [
        pltpu.async_copy(px_hbm.at[i0_v], p0x_v, sem),
        pltpu.async_copy(py_hbm.at[i0_v], p0y_v, sem),
        pltpu.async_copy(vx_hbm.at[i0_v], v0x_v, sem),
        pltpu.async_copy(vy_hbm.at[i0_v], v0y_v, sem),
        pltpu.async_copy(px_hbm.at[i1_v], p1x_v, sem),
        pltpu.async_copy(vx_hbm.at[i1_v], v1x_v, sem),
        pltpu.async_copy(py_hbm.at[i2_v], p2y_v, sem),
        pltpu.async_copy(vy_hbm.at[i2_v], v2y_v, sem),
    ]
    for c in copies:
        c.wait()
    dxs = p1x_v[...] - p0x_v[...]
    dys = p2y_v[...] - p0y_v[...]
    dvx = v1x_v[...] - v0x_v[...]
    dvy = v2y_v[...] - v0y_v[...]
    div = dvx / (dxs + 1e-6) + dvy / (dys + 1e-6)
    res_v[...] = div * div
    pltpu.sync_copy(res_v, out_hbm.at[wid])


def _loss_body(rows_ref, cols_ref, obs_ref, out_ref):
    i = pl.program_id(1)
    rows = rows_ref[0]
    px_i = rows[:, 0:1]
    py_i = rows[:, 1:2]
    vx_i = rows[:, 2:3]
    vy_i = rows[:, 3:4]
    tx_i = rows[:, 4:5]
    ty_i = rows[:, 5:6]
    msk = rows[:, 6:7]

    cols = cols_ref[0]
    px_j = cols[0:1, :]
    py_j = cols[1:2, :]
    vx_j = cols[2:3, :]
    vy_j = cols[3:4, :]

    dx = px_i - px_j
    dy = py_i - py_j
    dsq = dx * dx + dy * dy
    wx = vx_i - vx_j
    wy = vy_i - vy_j
    vsq = wx * wx + wy * wy

    col_ids = jax.lax.broadcasted_iota(jnp.int32, (1, _N), 1)
    row_ids = i * _R + jax.lax.broadcasted_iota(jnp.int32, (_R, 1), 0)
    idmask = jnp.int32(-2048)
    kbits = (jax.lax.bitcast_convert_type(dsq, jnp.int32) & idmask) | col_ids
    bigbits = jnp.int32(0x7F000000)
    kbits = jnp.where(col_ids == row_ids, bigbits, kbits)
    keys = jax.lax.bitcast_convert_type(kbits, jnp.float32)
    bigkey = jax.lax.bitcast_convert_type(bigbits, jnp.float32)

    acc = jnp.zeros((_R, 1), jnp.float32)
    for _ in range(_K):
        kmin = jnp.min(keys, axis=1, keepdims=True)
        sel = keys == kmin
        vsel = jnp.sum(jnp.where(sel, vsq, 0.0), axis=1, keepdims=True)
        vd = jnp.sqrt(vsel + 1e-12)
        dsqmin = jax.lax.bitcast_convert_type(
            jax.lax.bitcast_convert_type(kmin, jnp.int32) & idmask,
            jnp.float32)
        dmin = jnp.sqrt(dsqmin + 1e-12)
        acc = acc + vd * (1.0 / (dmin + 1e-6))
        keys = jnp.where(sel, bigkey, keys)
    con_part = jnp.sum(acc)

    vl_part = jnp.sum((vx_i - tx_i) ** 2 + (vy_i - ty_i) ** 2)

    a0, a1, a2, a3 = px_i, 1.0 - px_i, py_i, 1.0 - py_i
    is0 = (a0 <= a1) & (a0 <= a2) & (a0 <= a3)
    is1 = (~is0) & (a1 <= a2) & (a1 <= a3)
    is2 = (~is0) & (~is1) & (a2 <= a3)
    is3 = (~is0) & (~is1) & (~is2)
    nx = jnp.where(is0, -1.0, jnp.where(is1, 1.0, 0.0))
    ny = jnp.where(is2, -1.0, jnp.where(is3, 1.0, 0.0))
    nc = vx_i * nx + vy_i * ny
    bl_num = jnp.sum(nc * nc * msk)
    bl_cnt = jnp.sum(msk)

    cx = obs_ref[0, 0:1, :]
    cy = obs_ref[0, 1:2, :]
    rr = obs_ref[0, 2:3, :]
    dxo = px_i - cx
    dyo = py_i - cy
    disto = jnp.sqrt(dxo * dxo + dyo * dyo + 1e-12)
    near = (disto < rr * 2.0).astype(jnp.float32)
    wexp = jnp.exp(-(disto - rr) / (rr * 0.5))
    proj = (vx_i * dxo + vy_i * dyo) / (disto + 1e-6)
    pen = wexp * jnp.maximum(-proj, 0.0) ** 2
    pns = jnp.sum(pen * near, axis=0, keepdims=True)
    ncnt = jnp.sum(near, axis=0, keepdims=True)

    def bc(s):
        return jnp.broadcast_to(jnp.reshape(s, (1, 1)), (1, 128))

    tile = jnp.concatenate(
        [bc(con_part), bc(vl_part), bc(bl_num), bc(bl_cnt),
         pns, ncnt, jnp.zeros((2, 128), jnp.float32)], axis=0)
    out_ref[0, 0] = tile


@jax.jit
def kernel(predicted_velocities, target_velocities, positions, obstacles,
           boundary_mask):
    px = positions[..., 0].reshape(-1)
    py = positions[..., 1].reshape(-1)
    vx = predicted_velocities[..., 0].reshape(-1)
    vy = predicted_velocities[..., 1].reshape(-1)
    idx0 = jnp.asarray(_DIV_IDX[0])
    idx1 = jnp.asarray(_DIV_IDX[1])
    idx2 = jnp.asarray(_DIV_IDX[2])
    div_sq = _sc_divergence(px, py, vx, vy, idx0, idx1, idx2)
    dl = div_sq.reshape(-1)[: _B * _S].sum() / (_B * _S)

    mask_f = boundary_mask.astype(jnp.float32)[..., None]
    zeros_rows = jnp.zeros((_B, _N, 1), jnp.float32)
    rows = jnp.concatenate(
        [positions, predicted_velocities, target_velocities, mask_f,
         zeros_rows], axis=-1)
    cols = jnp.concatenate(
        [jnp.transpose(positions, (0, 2, 1)),
         jnp.transpose(predicted_velocities, (0, 2, 1)),
         jnp.zeros((_B, 4, _N), jnp.float32)], axis=1)
    obs_p = jnp.zeros((_B, 8, _OBW), jnp.float32)
    obs_p = obs_p.at[:, 0:3, 0:_M].set(jnp.transpose(obstacles, (0, 2, 1)))

    parts = pl.pallas_call(
        _loss_body,
        grid=(_B, _NB),
        in_specs=[
            pl.BlockSpec((1, _R, 8), lambda b, i: (b, i, 0)),
            pl.BlockSpec((1, 8, _N), lambda b, i: (b, 0, 0)),
            pl.BlockSpec((1, 8, _OBW), lambda b, i: (b, 0, 0)),
        ],
        out_specs=pl.BlockSpec((1, 1, 8, 128), lambda b, i: (b, i, 0, 0)),
        out_shape=jax.ShapeDtypeStruct((_B, _NB, 8, 128), jnp.float32),
    )(rows, cols, obs_p)

    cl = parts[:, :, 0, 0].sum() / (_B * _N * _K)
    vl = parts[:, :, 1, 0].sum() / (_B * _N * 2)
    bln = parts[:, :, 2, 0].sum()
    blc = parts[:, :, 3, 0].sum()
    bl = jnp.where(blc > 0, bln / jnp.maximum(blc, 1.0), 0.0)
    pns = parts[:, :, 4, 0:_M].sum(axis=1)
    cnt = parts[:, :, 5, 0:_M].sum(axis=1)
    rr = obstacles[:, :, 2]
    penalty = jnp.where(cnt > 0, pns / jnp.maximum(cnt, 1.0), 0.0)
    penalty = penalty * (rr > 0).astype(jnp.float32)
    ol = penalty.sum() / (_B * _M)

    return (_VEL_W * vl + _CON_W * cl + _BND_W * bl + _OBS_W * ol
            + _DIV_W * dl)


# speedup vs baseline: 1.5725x; 1.2247x over previous
import functools

import jax
import jax.numpy as jnp
import numpy as np
from jax.experimental import pallas as pl
from jax.experimental.pallas import tpu as pltpu
from jax.experimental.pallas import tpu_sc as plsc

_VEL_W, _CON_W, _BND_W, _OBS_W, _DIV_W = 1.0, 0.1, 0.5, 1.0, 0.1
_B, _N, _M = 4, 2048, 16
_K = 5
_R = 512
_NB = _N // _R
_S = 100
_OBW = 128
_NW = 32
_LANES = 16
_GPAD = _NW * _LANES
_RPW = (_B * _N) // _NW
_NGV = (_RPW * _K) // 128


def _div_indices():
    rng = np.random.default_rng(0)
    idx = np.stack([rng.permutation(_N)[:4] for _ in range(_S)])
    gidx = np.zeros((3, _GPAD), np.int32)
    for g in range(_B * _S):
        b, s = divmod(g, _S)
        for j in range(3):
            gidx[j, g] = b * _N + idx[s, j]
    return gidx


_DIV_IDX = _div_indices()

_sc_mesh = plsc.VectorSubcoreMesh(core_axis_name="c", subcore_axis_name="s")


def _nsqrt(x):
    i = jax.lax.bitcast_convert_type(x, jnp.int32)
    seed = jnp.int32(0x1FBD1DF5) + jax.lax.shift_right_arithmetic(i, 1)
    y = jax.lax.bitcast_convert_type(seed, jnp.float32)
    y = 0.5 * (y + x / y)
    y = 0.5 * (y + x / y)
    y = 0.5 * (y + x / y)
    return y


@functools.partial(
    pl.kernel,
    mesh=_sc_mesh,
    out_type=[
        jax.ShapeDtypeStruct((_NW, _LANES), jnp.float32),
        jax.ShapeDtypeStruct((_NW, _LANES), jnp.float32),
    ],
    scratch_types=[
        pltpu.VMEM((_LANES,), jnp.int32),
        pltpu.VMEM((_LANES,), jnp.int32),
        pltpu.VMEM((_LANES,), jnp.int32),
        pltpu.VMEM((_LANES,), jnp.float32),
        pltpu.VMEM((_LANES,), jnp.float32),
        pltpu.VMEM((_LANES,), jnp.float32),
        pltpu.VMEM((_LANES,), jnp.float32),
        pltpu.VMEM((_LANES,), jnp.float32),
        pltpu.VMEM((_LANES,), jnp.float32),
        pltpu.VMEM((_LANES,), jnp.float32),
        pltpu.VMEM((_LANES,), jnp.float32),
        pltpu.VMEM((_LANES,), jnp.float32),
        pltpu.VMEM((_NGV, 128), jnp.int32),
        pltpu.VMEM((_NGV, 128), jnp.float32),
        pltpu.VMEM((_RPW,), jnp.float32),
        pltpu.VMEM((_RPW,), jnp.float32),
        pltpu.VMEM((_NGV, 128), jnp.int32),
        pltpu.VMEM((_NGV, 128), jnp.float32),
        pltpu.VMEM((_NGV, 128), jnp.float32),
        pltpu.VMEM((_NGV, 128), jnp.float32),
        pltpu.VMEM((_LANES,), jnp.float32),
        pltpu.SemaphoreType.DMA,
    ],
)
def _sc_gather(px_hbm, py_hbm, vx_hbm, vy_hbm, idx0_hbm, idx1_hbm,
               idx2_hbm, keys_hbm, div_hbm, con_hbm, i0_v, i1_v, i2_v,
               p0x_v, p0y_v, v0x_v, v0y_v, p1x_v, v1x_v, p2y_v, v2y_v,
               res_v, kidx_v, kkey_v, vxo_v, vyo_v, nidx_v, ndsq_v, nvx_v,
               nvy_v, cres_v, sem):
    wid = jax.lax.axis_index("s") * 2 + jax.lax.axis_index("c")

    base = wid * _LANES
    pltpu.sync_copy(idx0_hbm.at[pl.ds(base, _LANES)], i0_v)
    pltpu.sync_copy(idx1_hbm.at[pl.ds(base, _LANES)], i1_v)
    pltpu.sync_copy(idx2_hbm.at[pl.ds(base, _LANES)], i2_v)
    copies = [
        pltpu.async_copy(px_hbm.at[i0_v], p0x_v, sem),
        pltpu.async_copy(py_hbm.at[i0_v], p0y_v, sem),
        pltpu.async_copy(vx_hbm.at[i0_v], v0x_v, sem),
        pltpu.async_copy(vy_hbm.at[i0_v], v0y_v, sem),
        pltpu.async_copy(px_hbm.at[i1_v], p1x_v, sem),
        pltpu.async_copy(vx_hbm.at[i1_v], v1x_v, sem),
        pltpu.async_copy(py_hbm.at[i2_v], p2y_v, sem),
        pltpu.async_copy(vy_hbm.at[i2_v], v2y_v, sem),
    ]
    for c in copies:
        c.wait()
    dxs = p1x_v[...] - p0x_v[...]
    dys = p2y_v[...] - p0y_v[...]
    dvx = v1x_v[...] - v0x_v[...]
    dvy = v2y_v[...] - v0y_v[...]
    div = dvx / (dxs + 1e-6) + dvy / (dys + 1e-6)
    res_v[...] = div * div
    pltpu.sync_copy(res_v, div_hbm.at[wid])

    rowbase = wid * _RPW
    bflat = (wid // (_N // _RPW)) * _N
    pltpu.sync_copy(vx_hbm.at[pl.ds(rowbase, _RPW)], vxo_v)
    pltpu.sync_copy(vy_hbm.at[pl.ds(rowbase, _RPW)], vyo_v)

    lane8 = jax.lax.iota(jnp.int32, _LANES) * 8
    kbase = rowbase * 8
    for k in range(_K):
        for j in range(_RPW // _LANES):
            p = k * _RPW + j * _LANES
            g, o = divmod(p, 128)
            kidx_v[g, pl.ds(o, _LANES)] = lane8 + (kbase + j * 128 + k)
    kcopies = [
        pltpu.async_copy(keys_hbm.at[kidx_v.at[g]], kkey_v.at[g], sem)
        for g in range(_NGV)
    ]
    for c in kcopies:
        c.wait()

    idmask = jnp.int32(-2048)
    for g in range(_NGV):
        for o in range(0, 128, _LANES):
            kv = kkey_v[g, pl.ds(o, _LANES)]
            kb = jax.lax.bitcast_convert_type(kv, jnp.int32)
            colid = kb & jnp.int32(2047)
            dsqt = jax.lax.bitcast_convert_type(kb & idmask, jnp.float32)
            nidx_v[g, pl.ds(o, _LANES)] = colid + bflat
            ndsq_v[g, pl.ds(o, _LANES)] = dsqt

    gcopies = []
    for g in range(_NGV):
        gcopies.append(
            pltpu.async_copy(vx_hbm.at[nidx_v.at[g]], nvx_v.at[g], sem))
        gcopies.append(
            pltpu.async_copy(vy_hbm.at[nidx_v.at[g]], nvy_v.at[g], sem))
    for c in gcopies:
        c.wait()

    acc = jnp.zeros((_LANES,), jnp.float32)
    for k in range(_K):
        for j in range(_RPW // _LANES):
            p = k * _RPW + j * _LANES
            g, o = divmod(p, 128)
            vxn = nvx_v[g, pl.ds(o, _LANES)]
            vyn = nvy_v[g, pl.ds(o, _LANES)]
            dsqt = ndsq_v[g, pl.ds(o, _LANES)]
            vxi = vxo_v[pl.ds(j * _LANES, _LANES)]
            vyi = vyo_v[pl.ds(j * _LANES, _LANES)]
            wx = vxi - vxn
            wy = vyi - vyn
            vd = _nsqrt(wx * wx + wy * wy + 1e-12)
            dmin = _nsqrt(dsqt + 1e-12)
            acc = acc + vd * (1.0 / (dmin + 1e-6))
    cres_v[...] = acc
    pltpu.sync_copy(cres_v, con_hbm.at[wid])


def _loss_body(rows_ref, cols_ref, obs_ref, out_ref, keys_ref):
    i = pl.program_id(1)
    rows = rows_ref[0]
    px_i = rows[:, 0:1]
    py_i = rows[:, 1:2]
    vx_i = rows[:, 2:3]
    vy_i = rows[:, 3:4]
    tx_i = rows[:, 4:5]
    ty_i = rows[:, 5:6]
    msk = rows[:, 6:7]

    cols = cols_ref[0]
    px_j = cols[0:1, :]
    py_j = cols[1:2, :]

    dx = px_i - px_j
    dy = py_i - py_j
    dsq = dx * dx + dy * dy

    col_ids = jax.lax.broadcasted_iota(jnp.int32, (1, _N), 1)
    row_ids = i * _R + jax.lax.broadcasted_iota(jnp.int32, (_R, 1), 0)
    idmask = jnp.int32(-2048)
    kbits = (jax.lax.bitcast_convert_type(dsq, jnp.int32) & idmask) | col_ids
    bigbits = jnp.int32(0x7F000000)
    kbits = jnp.where(col_ids == row_ids, bigbits, kbits)
    keys = jax.lax.bitcast_convert_type(kbits, jnp.float32)
    bigkey = jax.lax.bitcast_convert_type(bigbits, jnp.float32)

    kmins = []
    for _ in range(_K):
        kmin = jnp.min(keys, axis=1, keepdims=True)
        sel = keys == kmin
        kmins.append(kmin)
        keys = jnp.where(sel, bigkey, keys)
    kmins.append(jnp.zeros((_R, 3), jnp.float32))
    keys_ref[0] = jnp.concatenate(kmins, axis=1)

    vl_part = jnp.sum((vx_i - tx_i) ** 2 + (vy_i - ty_i) ** 2)

    a0, a1, a2, a3 = px_i, 1.0 - px_i, py_i, 1.0 - py_i
    is0 = (a0 <= a1) & (a0 <= a2) & (a0 <= a3)
    is1 = (~is0) & (a1 <= a2) & (a1 <= a3)
    is2 = (~is0) & (~is1) & (a2 <= a3)
    is3 = (~is0) & (~is1) & (~is2)
    nx = jnp.where(is0, -1.0, jnp.where(is1, 1.0, 0.0))
    ny = jnp.where(is2, -1.0, jnp.where(is3, 1.0, 0.0))
    nc = vx_i * nx + vy_i * ny
    bl_num = jnp.sum(nc * nc * msk)
    bl_cnt = jnp.sum(msk)

    cx = obs_ref[0, 0:1, :]
    cy = obs_ref[0, 1:2, :]
    rr = obs_ref[0, 2:3, :]
    dxo = px_i - cx
    dyo = py_i - cy
    disto = jnp.sqrt(dxo * dxo + dyo * dyo + 1e-12)
    near = (disto < rr * 2.0).astype(jnp.float32)
    wexp = jnp.exp(-(disto - rr) / (rr * 0.5))
    proj = (vx_i * dxo + vy_i * dyo) / (disto + 1e-6)
    pen = wexp * jnp.maximum(-proj, 0.0) ** 2
    pns = jnp.sum(pen * near, axis=0, keepdims=True)
    ncnt = jnp.sum(near, axis=0, keepdims=True)

    def bc(s):
        return jnp.broadcast_to(jnp.reshape(s, (1, 1)), (1, 128))

    tile = jnp.concatenate(
        [bc(vl_part), bc(bl_num), bc(bl_cnt),
         pns, ncnt, jnp.zeros((3, 128), jnp.float32)], axis=0)
    out_ref[0, 0] = tile


@jax.jit
def kernel(predicted_velocities, target_velocities, positions, obstacles,
           boundary_mask):
    px = positions[..., 0].reshape(-1)
    py = positions[..., 1].reshape(-1)
    vx = predicted_velocities[..., 0].reshape(-1)
    vy = predicted_velocities[..., 1].reshape(-1)
    idx0 = jnp.asarray(_DIV_IDX[0])
    idx1 = jnp.asarray(_DIV_IDX[1])
    idx2 = jnp.asarray(_DIV_IDX[2])

    mask_f = boundary_mask.astype(jnp.float32)[..., None]
    zeros_rows = jnp.zeros((_B, _N, 1), jnp.float32)
    rows = jnp.concatenate(
        [positions, predicted_velocities, target_velocities, mask_f,
         zeros_rows], axis=-1)
    cols = jnp.concatenate(
        [jnp.transpose(positions, (0, 2, 1)),
         jnp.zeros((_B, 6, _N), jnp.float32)], axis=1)
    obs_p = jnp.zeros((_B, 8, _OBW), jnp.float32)
    obs_p = obs_p.at[:, 0:3, 0:_M].set(jnp.transpose(obstacles, (0, 2, 1)))

    parts, keys = pl.pallas_call(
        _loss_body,
        grid=(_B, _NB),
        in_specs=[
            pl.BlockSpec((1, _R, 8), lambda b, i: (b, i, 0)),
            pl.BlockSpec((1, 8, _N), lambda b, i: (b, 0, 0)),
            pl.BlockSpec((1, 8, _OBW), lambda b, i: (b, 0, 0)),
        ],
        out_specs=[
            pl.BlockSpec((1, 1, 8, 128), lambda b, i: (b, i, 0, 0)),
            pl.BlockSpec((1, _R, 8), lambda b, i: (b, i, 0)),
        ],
        out_shape=[
            jax.ShapeDtypeStruct((_B, _NB, 8, 128), jnp.float32),
            jax.ShapeDtypeStruct((_B, _N, 8), jnp.float32),
        ],
    )(rows, cols, obs_p)

    div_sq, con_parts = _sc_gather(px, py, vx, vy, idx0, idx1, idx2,
                                   keys.reshape(_B * _N * 8))
    dl = div_sq.reshape(-1)[: _B * _S].sum() / (_B * _S)
    cl = con_parts.sum() / (_B * _N * _K)

    vl = parts[:, :, 0, 0].sum() / (_B * _N * 2)
    bln = parts[:, :, 1, 0].sum()
    blc = parts[:, :, 2, 0].sum()
    bl = jnp.where(blc > 0, bln / jnp.maximum(blc, 1.0), 0.0)
    pns = parts[:, :, 3, 0:_M].sum(axis=1)
    cnt = parts[:, :, 4, 0:_M].sum(axis=1)
    rr = obstacles[:, :, 2]
    penalty = jnp.where(cnt > 0, pns / jnp.maximum(cnt, 1.0), 0.0)
    penalty = penalty * (rr > 0).astype(jnp.float32)
    ol = penalty.sum() / (_B * _M)

    return (_VEL_W * vl + _CON_W * cl + _BND_W * bl + _OBS_W * ol
            + _DIV_W * dl)


# speedup vs baseline: 1.5830x; 1.0066x over previous
import functools

import jax
import jax.numpy as jnp
import numpy as np
from jax.experimental import pallas as pl
from jax.experimental.pallas import tpu as pltpu
from jax.experimental.pallas import tpu_sc as plsc

_VEL_W, _CON_W, _BND_W, _OBS_W, _DIV_W = 1.0, 0.1, 0.5, 1.0, 0.1
_B, _N, _M = 4, 2048, 16
_K = 5
_R = 1024
_NB = _N // _R
_S = 100
_OBW = 128
_NW = 32
_LANES = 16
_GPAD = _NW * _LANES
_RPW = (_B * _N) // _NW
_NGV = (_RPW * _K) // 128


def _div_indices():
    rng = np.random.default_rng(0)
    idx = np.stack([rng.permutation(_N)[:4] for _ in range(_S)])
    gidx = np.zeros((3, _GPAD), np.int32)
    for g in range(_B * _S):
        b, s = divmod(g, _S)
        for j in range(3):
            gidx[j, g] = b * _N + idx[s, j]
    return gidx


_DIV_IDX = _div_indices()

_sc_mesh = plsc.VectorSubcoreMesh(core_axis_name="c", subcore_axis_name="s")


def _nsqrt(x):
    i = jax.lax.bitcast_convert_type(x, jnp.int32)
    seed = jnp.int32(0x1FBD1DF5) + jax.lax.shift_right_arithmetic(i, 1)
    y = jax.lax.bitcast_convert_type(seed, jnp.float32)
    y = 0.5 * (y + x / y)
    y = 0.5 * (y + x / y)
    y = 0.5 * (y + x / y)
    return y


@functools.partial(
    pl.kernel,
    mesh=_sc_mesh,
    out_type=[
        jax.ShapeDtypeStruct((_NW, _LANES), jnp.float32),
        jax.ShapeDtypeStruct((_NW, _LANES), jnp.float32),
    ],
    scratch_types=[
        pltpu.VMEM((_LANES,), jnp.int32),
        pltpu.VMEM((_LANES,), jnp.int32),
        pltpu.VMEM((_LANES,), jnp.int32),
        pltpu.VMEM((_LANES,), jnp.float32),
        pltpu.VMEM((_LANES,), jnp.float32),
        pltpu.VMEM((_LANES,), jnp.float32),
        pltpu.VMEM((_LANES,), jnp.float32),
        pltpu.VMEM((_LANES,), jnp.float32),
        pltpu.VMEM((_LANES,), jnp.float32),
        pltpu.VMEM((_LANES,), jnp.float32),
        pltpu.VMEM((_LANES,), jnp.float32),
        pltpu.VMEM((_LANES,), jnp.float32),
        pltpu.VMEM((_NGV, 128), jnp.int32),
        pltpu.VMEM((_NGV, 128), jnp.float32),
        pltpu.VMEM((_RPW,), jnp.float32),
        pltpu.VMEM((_RPW,), jnp.float32),
        pltpu.VMEM((_NGV, 128), jnp.int32),
        pltpu.VMEM((_NGV, 128), jnp.float32),
        pltpu.VMEM((_NGV, 128), jnp.float32),
        pltpu.VMEM((_NGV, 128), jnp.float32),
        pltpu.VMEM((_LANES,), jnp.float32),
        pltpu.SemaphoreType.DMA,
    ],
)
def _sc_gather(px_hbm, py_hbm, vx_hbm, vy_hbm, idx0_hbm, idx1_hbm,
               idx2_hbm, keys_hbm, div_hbm, con_hbm, i0_v, i1_v, i2_v,
               p0x_v, p0y_v, v0x_v, v0y_v, p1x_v, v1x_v, p2y_v, v2y_v,
               res_v, kidx_v, kkey_v, vxo_v, vyo_v, nidx_v, ndsq_v, nvx_v,
               nvy_v, cres_v, sem):
    wid = jax.lax.axis_index("s") * 2 + jax.lax.axis_index("c")

    base = wid * _LANES
    pltpu.sync_copy(idx0_hbm.at[pl.ds(base, _LANES)], i0_v)
    pltpu.sync_copy(idx1_hbm.at[pl.ds(base, _LANES)], i1_v)
    pltpu.sync_copy(idx2_hbm.at[pl.ds(base, _LANES)], i2_v)
    copies = [
        pltpu.async_copy(px_hbm.at[i0_v], p0x_v, sem),
        pltpu.async_copy(py_hbm.at[i0_v], p0y_v, sem),
        pltpu.async_copy(vx_hbm.at[i0_v], v0x_v, sem),
        pltpu.async_copy(vy_hbm.at[i0_v], v0y_v, sem),
        pltpu.async_copy(px_hbm.at[i1_v], p1x_v, sem),
        pltpu.async_copy(vx_hbm.at[i1_v], v1x_v, sem),
        pltpu.async_copy(py_hbm.at[i2_v], p2y_v, sem),
        pltpu.async_copy(vy_hbm.at[i2_v], v2y_v, sem),
    ]
    for c in copies:
        c.wait()
    dxs = p1x_v[...] - p0x_v[...]
    dys = p2y_v[...] - p0y_v[...]
    dvx = v1x_v[...] - v0x_v[...]
    dvy = v2y_v[...] - v0y_v[...]
    div = dvx / (dxs + 1e-6) + dvy / (dys + 1e-6)
    res_v[...] = div * div
    pltpu.sync_copy(res_v, div_hbm.at[wid])

    rowbase = wid * _RPW
    bflat = (wid // (_N // _RPW)) * _N
    pltpu.sync_copy(vx_hbm.at[pl.ds(rowbase, _RPW)], vxo_v)
    pltpu.sync_copy(vy_hbm.at[pl.ds(rowbase, _RPW)], vyo_v)

    lane8 = jax.lax.iota(jnp.int32, _LANES) * 8
    kbase = rowbase * 8
    for k in range(_K):
        for j in range(_RPW // _LANES):
            p = k * _RPW + j * _LANES
            g, o = divmod(p, 128)
            kidx_v[g, pl.ds(o, _LANES)] = lane8 + (kbase + j * 128 + k)
    kcopies = [
        pltpu.async_copy(keys_hbm.at[kidx_v.at[g]], kkey_v.at[g], sem)
        for g in range(_NGV)
    ]
    for c in kcopies:
        c.wait()

    idmask = jnp.int32(-2048)
    for g in range(_NGV):
        for o in range(0, 128, _LANES):
            kv = kkey_v[g, pl.ds(o, _LANES)]
            kb = jax.lax.bitcast_convert_type(kv, jnp.int32)
            colid = kb & jnp.int32(2047)
            dsqt = jax.lax.bitcast_convert_type(kb & idmask, jnp.float32)
            nidx_v[g, pl.ds(o, _LANES)] = colid + bflat
            ndsq_v[g, pl.ds(o, _LANES)] = dsqt

    gcopies = []
    for g in range(_NGV):
        gcopies.append(
            pltpu.async_copy(vx_hbm.at[nidx_v.at[g]], nvx_v.at[g], sem))
        gcopies.append(
            pltpu.async_copy(vy_hbm.at[nidx_v.at[g]], nvy_v.at[g], sem))
    for c in gcopies:
        c.wait()

    acc = jnp.zeros((_LANES,), jnp.float32)
    for k in range(_K):
        for j in range(_RPW // _LANES):
            p = k * _RPW + j * _LANES
            g, o = divmod(p, 128)
            vxn = nvx_v[g, pl.ds(o, _LANES)]
            vyn = nvy_v[g, pl.ds(o, _LANES)]
            dsqt = ndsq_v[g, pl.ds(o, _LANES)]
            vxi = vxo_v[pl.ds(j * _LANES, _LANES)]
            vyi = vyo_v[pl.ds(j * _LANES, _LANES)]
            wx = vxi - vxn
            wy = vyi - vyn
            vd = _nsqrt(wx * wx + wy * wy + 1e-12)
            dmin = _nsqrt(dsqt + 1e-12)
            acc = acc + vd * (1.0 / (dmin + 1e-6))
    cres_v[...] = acc
    pltpu.sync_copy(cres_v, con_hbm.at[wid])


def _loss_body(rows_ref, cols_ref, obs_ref, out_ref, keys_ref):
    i = pl.program_id(1)
    rows = rows_ref[0]
    px_i = rows[:, 0:1]
    py_i = rows[:, 1:2]
    vx_i = rows[:, 2:3]
    vy_i = rows[:, 3:4]
    tx_i = rows[:, 4:5]
    ty_i = rows[:, 5:6]
    msk = rows[:, 6:7]

    cols = cols_ref[0]
    px_j = cols[0:1, :]
    py_j = cols[1:2, :]

    dx = px_i - px_j
    dy = py_i - py_j
    dsq = dx * dx + dy * dy

    col_ids = jax.lax.broadcasted_iota(jnp.int32, (1, _N), 1)
    row_ids = i * _R + jax.lax.broadcasted_iota(jnp.int32, (_R, 1), 0)
    idmask = jnp.int32(-2048)
    kbits = (jax.lax.bitcast_convert_type(dsq, jnp.int32) & idmask) | col_ids
    bigbits = jnp.int32(0x7F000000)
    kbits = jnp.where(col_ids == row_ids, bigbits, kbits)
    keys = jax.lax.bitcast_convert_type(kbits, jnp.float32)
    bigkey = jax.lax.bitcast_convert_type(bigbits, jnp.float32)

    kmins = []
    for _ in range(_K):
        kmin = jnp.min(keys, axis=1, keepdims=True)
        sel = keys == kmin
        kmins.append(kmin)
        keys = jnp.where(sel, bigkey, keys)
    kmins.append(jnp.zeros((_R, 3), jnp.float32))
    keys_ref[0] = jnp.concatenate(kmins, axis=1)

    vl_part = jnp.sum((vx_i - tx_i) ** 2 + (vy_i - ty_i) ** 2)

    a0, a1, a2, a3 = px_i, 1.0 - px_i, py_i, 1.0 - py_i
    is0 = (a0 <= a1) & (a0 <= a2) & (a0 <= a3)
    is1 = (~is0) & (a1 <= a2) & (a1 <= a3)
    is2 = (~is0) & (~is1) & (a2 <= a3)
    is3 = (~is0) & (~is1) & (~is2)
    nx = jnp.where(is0, -1.0, jnp.where(is1, 1.0, 0.0))
    ny = jnp.where(is2, -1.0, jnp.where(is3, 1.0, 0.0))
    nc = vx_i * nx + vy_i * ny
    bl_num = jnp.sum(nc * nc * msk)
    bl_cnt = jnp.sum(msk)

    cx = obs_ref[0, 0:1, :]
    cy = obs_ref[0, 1:2, :]
    rr = obs_ref[0, 2:3, :]
    dxo = px_i - cx
    dyo = py_i - cy
    disto = jnp.sqrt(dxo * dxo + dyo * dyo + 1e-12)
    near = (disto < rr * 2.0).astype(jnp.float32)
    wexp = jnp.exp(-(disto - rr) / (rr * 0.5))
    proj = (vx_i * dxo + vy_i * dyo) / (disto + 1e-6)
    pen = wexp * jnp.maximum(-proj, 0.0) ** 2
    pns = jnp.sum(pen * near, axis=0, keepdims=True)
    ncnt = jnp.sum(near, axis=0, keepdims=True)

    def bc(s):
        return jnp.broadcast_to(jnp.reshape(s, (1, 1)), (1, 128))

    tile = jnp.concatenate(
        [bc(vl_part), bc(bl_num), bc(bl_cnt),
         pns, ncnt, jnp.zeros((3, 128), jnp.float32)], axis=0)
    out_ref[0, 0] = tile


@jax.jit
def kernel(predicted_velocities, target_velocities, positions, obstacles,
           boundary_mask):
    px = positions[..., 0].reshape(-1)
    py = positions[..., 1].reshape(-1)
    vx = predicted_velocities[..., 0].reshape(-1)
    vy = predicted_velocities[..., 1].reshape(-1)
    idx0 = jnp.asarray(_DIV_IDX[0])
    idx1 = jnp.asarray(_DIV_IDX[1])
    idx2 = jnp.asarray(_DIV_IDX[2])

    mask_f = boundary_mask.astype(jnp.float32)[..., None]
    zeros_rows = jnp.zeros((_B, _N, 1), jnp.float32)
    rows = jnp.concatenate(
        [positions, predicted_velocities, target_velocities, mask_f,
         zeros_rows], axis=-1)
    cols = jnp.concatenate(
        [jnp.transpose(positions, (0, 2, 1)),
         jnp.zeros((_B, 6, _N), jnp.float32)], axis=1)
    obs_p = jnp.zeros((_B, 8, _OBW), jnp.float32)
    obs_p = obs_p.at[:, 0:3, 0:_M].set(jnp.transpose(obstacles, (0, 2, 1)))

    parts, keys = pl.pallas_call(
        _loss_body,
        grid=(_B, _NB),
        in_specs=[
            pl.BlockSpec((1, _R, 8), lambda b, i: (b, i, 0)),
            pl.BlockSpec((1, 8, _N), lambda b, i: (b, 0, 0)),
            pl.BlockSpec((1, 8, _OBW), lambda b, i: (b, 0, 0)),
        ],
        out_specs=[
            pl.BlockSpec((1, 1, 8, 128), lambda b, i: (b, i, 0, 0)),
            pl.BlockSpec((1, _R, 8), lambda b, i: (b, i, 0)),
        ],
        out_shape=[
            jax.ShapeDtypeStruct((_B, _NB, 8, 128), jnp.float32),
            jax.ShapeDtypeStruct((_B, _N, 8), jnp.float32),
        ],
    )(rows, cols, obs_p)

    div_sq, con_parts = _sc_gather(px, py, vx, vy, idx0, idx1, idx2,
                                   keys.reshape(_B * _N * 8))
    dl = div_sq.reshape(-1)[: _B * _S].sum() / (_B * _S)
    cl = con_parts.sum() / (_B * _N * _K)

    vl = parts[:, :, 0, 0].sum() / (_B * _N * 2)
    bln = parts[:, :, 1, 0].sum()
    blc = parts[:, :, 2, 0].sum()
    bl = jnp.where(blc > 0, bln / jnp.maximum(blc, 1.0), 0.0)
    pns = parts[:, :, 3, 0:_M].sum(axis=1)
    cnt = parts[:, :, 4, 0:_M].sum(axis=1)
    rr = obstacles[:, :, 2]
    penalty = jnp.where(cnt > 0, pns / jnp.maximum(cnt, 1.0), 0.0)
    penalty = penalty * (rr > 0).astype(jnp.float32)
    ol = penalty.sum() / (_B * _M)

    return (_VEL_W * vl + _CON_W * cl + _BND_W * bl + _OBS_W * ol
            + _DIV_W * dl)
